# baseline (speedup 1.0000x reference)
import functools

import jax
import jax.numpy as jnp
from jax import lax
from jax.experimental import pallas as pl
from jax.experimental.pallas import tpu as pltpu
from jax.experimental.pallas import tpu_sc as plsc

N = 10000
E = 320000
D = 128
NCORE = 2
NSUB = 16
NW = NCORE * NSUB
CH = 128
NCHUNK = E // CH
NP = 10240
NPH = NP // 16
F32 = jnp.float32

_mesh = plsc.VectorSubcoreMesh(core_axis_name="c", subcore_axis_name="s")


def _worker(c, s):
    return s * NCORE + c


def _degs_body(dst_hbm, deg_hbm, idx_d, hist, semA):
    c = lax.axis_index("c")
    s = lax.axis_index("s")
    w = _worker(c, s)

    def zrow(i, _):
        hist[pl.ds(i * 16, 16)] = jnp.zeros((16,), F32)
        return 0

    lax.fori_loop(0, NP // 16, zrow, 0)
    nch = jnp.int32(NCHUNK // NW) + jnp.where(w < NCHUNK % NW, 1, 0).astype(jnp.int32)
    ones16 = jnp.ones((16,), F32)

    def chunk(k, _):
        base = (w + NW * k) * CH
        pltpu.async_copy(dst_hbm.at[pl.ds(base, CH)], idx_d, semA).wait()
        for g in range(8):
            idx16 = idx_d[pl.ds(g * 16, 16)]
            plsc.addupdate_scatter(hist, (idx16,), ones16)
        return 0

    lax.fori_loop(0, nch, chunk, 0)
    pltpu.async_copy(hist, deg_hbm.at[pl.ds(w * NP, NP)], semA).wait()


_degs = functools.partial(
    pl.kernel,
    out_type=jax.ShapeDtypeStruct((NW * NP,), F32),
    mesh=_mesh,
    compiler_params=pltpu.CompilerParams(needs_layout_passes=False),
    scratch_types=[
        pltpu.VMEM((CH,), jnp.int32),
        pltpu.VMEM((NP,), F32),
        pltpu.SemaphoreType.DMA,
    ],
)(_degs_body)


def _edge_body(ec_hbm, src_hbm, dst_hbm, tsrc_hbm, tdst_hbm,
               enew_hbm, msgt_hbm, est_hbm,
               idx_s, idx_d, ec_v, g_v, q_v, mt_v, st_v, sA, sB, sC):
    c = lax.axis_index("c")
    s = lax.axis_index("s")
    w = _worker(c, s)
    nch = jnp.int32(NCHUNK // NW) + jnp.where(w < NCHUNK % NW, 1, 0).astype(jnp.int32)
    zero16 = jnp.zeros((16,), F32)
    init = (zero16,) * 16
    iota16 = lax.broadcasted_iota(jnp.int32, (16,), 0)
    rows_g = [iota16 + g * 16 for g in range(8)]

    def chunk_a(k, accs):
        base = (w + NW * k) * CH
        pltpu.async_copy(src_hbm.at[pl.ds(base, CH)], idx_s, sA).wait()
        pltpu.async_copy(dst_hbm.at[pl.ds(base, CH)], idx_d, sB).wait()
        cp1 = pltpu.async_copy(ec_hbm.at[pl.ds(base, CH)], ec_v, sA)
        cp2 = pltpu.async_copy(tsrc_hbm.at[idx_s], g_v, sB)
        cp3 = pltpu.async_copy(tdst_hbm.at[idx_d], q_v, sC)
        cp1.wait()
        cp2.wait()
        cp3.wait()

        def row(r, a):
            col_r = jnp.full((16,), 0, jnp.int32) + r
            out = []
            for g in range(8):
                sl = pl.ds(g * 16, 16)
                en = ec_v[r, sl] + g_v[r, sl] + q_v[r, sl]
                ec_v[r, sl] = en
                sig = 1.0 / (1.0 + jnp.exp(-en))
                val = sig * g_v[r, pl.ds(D + g * 16, 16)]
                plsc.store_scatter(mt_v, (rows_g[g], col_r), val)
                out.append(a[g] + en)
            for g in range(8):
                sl = pl.ds(g * 16, 16)
                en = ec_v[r, sl]
                out.append(a[8 + g] + en * en)
            return tuple(out)

        accs = lax.fori_loop(0, CH, row, accs)
        pltpu.async_copy(ec_v, enew_hbm.at[pl.ds(base, CH)], sA).wait()
        pltpu.async_copy(mt_v, msgt_hbm.at[:, pl.ds(base, CH)], sB).wait()
        return accs

    accs = lax.fori_loop(0, nch, chunk_a, init)
    for g in range(8):
        st_v[0, pl.ds(g * 16, 16)] = accs[g]
        st_v[1, pl.ds(g * 16, 16)] = accs[8 + g]
    pltpu.async_copy(st_v, est_hbm.at[w], sC).wait()


_edge = functools.partial(
    pl.kernel,
    out_type=[
        jax.ShapeDtypeStruct((E, D), F32),
        jax.ShapeDtypeStruct((D, E), F32),
        jax.ShapeDtypeStruct((NW, 2, D), F32),
    ],
    mesh=_mesh,
    compiler_params=pltpu.CompilerParams(needs_layout_passes=False),
    scratch_types=[
        pltpu.VMEM((CH,), jnp.int32),
        pltpu.VMEM((CH,), jnp.int32),
        pltpu.VMEM((CH, D), F32),
        pltpu.VMEM((CH, 2 * D), F32),
        pltpu.VMEM((CH, D), F32),
        pltpu.VMEM((D, CH), F32),
        pltpu.VMEM((2, D), F32),
        pltpu.SemaphoreType.DMA,
        pltpu.SemaphoreType.DMA,
        pltpu.SemaphoreType.DMA,
    ],
)(_edge_body)


def _hagg_body(msgt_hbm, dst_hbm, hagg_hbm, idx_d, mt_v, hist, semA, semB):
    c = lax.axis_index("c")
    s = lax.axis_index("s")
    w = _worker(c, s)
    ch0 = (w // 2) * 8
    par = w % 2

    def zrow(i, _):
        for g in range(8):
            hist[i, pl.ds(g * 16, 16)] = jnp.zeros((16,), F32)
        return 0

    lax.fori_loop(0, NPH, zrow, 0)
    nhalf = NCHUNK // 2

    def chunk(k, _):
        kk = 2 * k + par
        base = kk * CH
        pltpu.async_copy(dst_hbm.at[pl.ds(base, CH)], idx_d, semA).wait()
        pltpu.async_copy(msgt_hbm.at[pl.ds(ch0, 8), pl.ds(base, CH)], mt_v, semB).wait()
        for g in range(8):
            dst16 = idx_d[pl.ds(g * 16, 16)]
            row16 = lax.shift_right_logical(dst16, 4)
            colb = lax.shift_left(jnp.bitwise_and(dst16, 15), 3)
            for c8 in range(8):
                val = mt_v[c8, pl.ds(g * 16, 16)]
                plsc.addupdate_scatter(hist, (row16, colb + c8), val)
        return 0

    lax.fori_loop(0, nhalf, chunk, 0)
    pltpu.async_copy(hist, hagg_hbm.at[w], semA).wait()


_hagg = functools.partial(
    pl.kernel,
    out_type=jax.ShapeDtypeStruct((NW, NPH, D), F32),
    mesh=_mesh,
    compiler_params=pltpu.CompilerParams(needs_layout_passes=False),
    scratch_types=[
        pltpu.VMEM((CH,), jnp.int32),
        pltpu.VMEM((8, CH), F32),
        pltpu.VMEM((NPH, D), F32),
        pltpu.SemaphoreType.DMA,
        pltpu.SemaphoreType.DMA,
    ],
)(_hagg_body)


def _k1_body(h_ref, wn_ref, wnb_ref, dg_ref, h0_ref, nm_ref):
    deg = jnp.maximum(jnp.sum(dg_ref[...], axis=0), 1.0)
    nm_ref[...] = lax.rsqrt(deg)
    h0_ref[...] = (
        jnp.dot(h_ref[...], wn_ref[...], preferred_element_type=F32) + wnb_ref[...]
    )


_k1 = pl.pallas_call(
    _k1_body,
    grid=(10,),
    in_specs=[
        pl.BlockSpec((1000, D), lambda i: (i, 0)),
        pl.BlockSpec((D, D), lambda i: (0, 0)),
        pl.BlockSpec((1, D), lambda i: (0, 0)),
        pl.BlockSpec((NW, 1000, 1), lambda i: (0, i, 0)),
    ],
    out_specs=[
        pl.BlockSpec((1000, D), lambda i: (i, 0)),
        pl.BlockSpec((1000, 1), lambda i: (i, 0)),
    ],
    out_shape=[
        jax.ShapeDtypeStruct((N, D), F32),
        jax.ShapeDtypeStruct((N, 1), F32),
    ],
)


def _k2_body(e_ref, we_ref, web_ref, cm_ref, cb_ref, e0_ref, ec_ref):
    e0 = jnp.dot(e_ref[...], we_ref[...], preferred_element_type=F32) + web_ref[...]
    e0_ref[...] = e0
    ec_ref[...] = jnp.dot(e0, cm_ref[...], preferred_element_type=F32) + cb_ref[...]


_k2 = pl.pallas_call(
    _k2_body,
    grid=(E // 512,),
    in_specs=[
        pl.BlockSpec((512, D), lambda i: (i, 0)),
        pl.BlockSpec((D, D), lambda i: (0, 0)),
        pl.BlockSpec((1, D), lambda i: (0, 0)),
        pl.BlockSpec((D, D), lambda i: (0, 0)),
        pl.BlockSpec((1, D), lambda i: (0, 0)),
    ],
    out_specs=[
        pl.BlockSpec((512, D), lambda i: (i, 0)),
        pl.BlockSpec((512, D), lambda i: (i, 0)),
    ],
    out_shape=[
        jax.ShapeDtypeStruct((E, D), F32),
        jax.ShapeDtypeStruct((E, D), F32),
    ],
)


def _knode_body(h_ref, nm_ref, a_ref, ab_ref, b_ref, bb_ref, d_ref, db_ref,
                em_ref, eb_ref, ah_ref, ts_ref, td_ref):
    hh = h_ref[...]
    ah_ref[...] = jnp.dot(hh, a_ref[...], preferred_element_type=F32) + ab_ref[...]
    dh = jnp.dot(hh, d_ref[...], preferred_element_type=F32) + db_ref[...]
    bh = (jnp.dot(hh, b_ref[...], preferred_element_type=F32) + bb_ref[...]) * nm_ref[...]
    ts_ref[:, 0:D] = dh
    ts_ref[:, D : 2 * D] = bh
    td_ref[...] = jnp.dot(hh, em_ref[...], preferred_element_type=F32) + eb_ref[...]


_knode = pl.pallas_call(
    _knode_body,
    grid=(10,),
    in_specs=[
        pl.BlockSpec((1000, D), lambda i: (i, 0)),
        pl.BlockSpec((1000, 1), lambda i: (i, 0)),
    ]
    + [
        spec
        for _ in range(4)
        for spec in (
            pl.BlockSpec((D, D), lambda i: (0, 0)),
            pl.BlockSpec((1, D), lambda i: (0, 0)),
        )
    ],
    out_specs=[
        pl.BlockSpec((1000, D), lambda i: (i, 0)),
        pl.BlockSpec((1000, 2 * D), lambda i: (i, 0)),
        pl.BlockSpec((1000, D), lambda i: (i, 0)),
    ],
    out_shape=[
        jax.ShapeDtypeStruct((N, D), F32),
        jax.ShapeDtypeStruct((N, 2 * D), F32),
        jax.ShapeDtypeStruct((N, D), F32),
    ],
)


def _kedge_body(en_ref, ep_ref, st_ref, ge_ref, be_ref, cm_ref, cb_ref,
                ei_ref, ec_ref):
    red = jnp.sum(st_ref[...], axis=0)
    mu = red[0] / E
    var = red[1] / E - mu * mu
    a = ge_ref[...] * lax.rsqrt(var + 1e-5)
    bb = be_ref[...] - mu * a
    ei = jax.nn.relu(en_ref[...] * a + bb) + ep_ref[...]
    ei_ref[...] = ei
    ec_ref[...] = jnp.dot(ei, cm_ref[...], preferred_element_type=F32) + cb_ref[...]


_kedge = pl.pallas_call(
    _kedge_body,
    grid=(E // 512,),
    in_specs=[
        pl.BlockSpec((512, D), lambda i: (i, 0)),
        pl.BlockSpec((512, D), lambda i: (i, 0)),
        pl.BlockSpec((NW, 2, D), lambda i: (0, 0, 0)),
        pl.BlockSpec((1, D), lambda i: (0, 0)),
        pl.BlockSpec((1, D), lambda i: (0, 0)),
        pl.BlockSpec((D, D), lambda i: (0, 0)),
        pl.BlockSpec((1, D), lambda i: (0, 0)),
    ],
    out_specs=[
        pl.BlockSpec((512, D), lambda i: (i, 0)),
        pl.BlockSpec((512, D), lambda i: (i, 0)),
    ],
    out_shape=[
        jax.ShapeDtypeStruct((E, D), F32),
        jax.ShapeDtypeStruct((E, D), F32),
    ],
)


def _knode2_body(ah_ref, hg_ref, nm_ref, h_ref, gh_ref, bhp_ref, o_ref):
    hn = ah_ref[...] + nm_ref[...] * (hg_ref[0] + hg_ref[1])
    mu = jnp.mean(hn, axis=0, keepdims=True)
    dlt = hn - mu
    var = jnp.mean(dlt * dlt, axis=0, keepdims=True)
    o_ref[...] = (
        jax.nn.relu(dlt * lax.rsqrt(var + 1e-5) * gh_ref[...] + bhp_ref[...])
        + h_ref[...]
    )


_knode2 = pl.pallas_call(
    _knode2_body,
    out_shape=jax.ShapeDtypeStruct((N, D), F32),
)


def _khead_body(h_ref, w1_ref, b1_ref, w2_ref, b2_ref, w3_ref, b3_ref, o_ref):
    hg = jnp.mean(h_ref[...], axis=0, keepdims=True)
    hg = jax.nn.relu(jnp.dot(hg, w1_ref[...], preferred_element_type=F32) + b1_ref[...])
    hg = jax.nn.relu(jnp.dot(hg, w2_ref[...], preferred_element_type=F32) + b2_ref[...])
    o_ref[...] = jnp.dot(hg, w3_ref[...], preferred_element_type=F32) + b3_ref[...]


def kernel(h, e, edge_index, Wn, Wn_b, We, We_b, A, A_b, Bm, B_b, Cm, C_b,
           Dm, D_b, Em, E_b, gh, bh, ge, be, W1, b1, W2, b2, W3, b3):
    src = edge_index[0]
    dst = edge_index[1]
    L = A.shape[0]

    deg_raw = _degs(dst)
    degs2 = deg_raw.reshape(NW, NP)[:, :N, None]
    h_cur, nm = _k1(h, Wn, Wn_b.reshape(1, D), degs2)
    e_cur, ec = _k2(e, We, We_b.reshape(1, D), Cm[0], C_b[0].reshape(1, D))
    enew = est = None
    for i in range(L):
        if i > 0:
            e_cur, ec = _kedge(
                enew, e_cur, est,
                ge[i - 1].reshape(1, D), be[i - 1].reshape(1, D),
                Cm[i], C_b[i].reshape(1, D),
            )
        ah, ts, td = _knode(
            h_cur, nm,
            A[i], A_b[i].reshape(1, D), Bm[i], B_b[i].reshape(1, D),
            Dm[i], D_b[i].reshape(1, D), Em[i], E_b[i].reshape(1, D),
        )
        enew, msgt, est = _edge(ec, src, dst, ts, td)
        hg = _hagg(msgt, dst)
        hgt = (hg.reshape(16, 2, NPH, 16, 8)
               .transpose(1, 2, 3, 0, 4)
               .reshape(2, NP, D)[:, :N])
        h_cur = _knode2(
            ah, hgt, nm, h_cur,
            gh[i].reshape(1, D), bh[i].reshape(1, D),
        )
    return pl.pallas_call(
        _khead_body,
        out_shape=jax.ShapeDtypeStruct((1, W3.shape[1]), F32),
    )(h_cur, W1, b1.reshape(1, -1), W2, b2.reshape(1, -1), W3, b3.reshape(1, -1))


# speedup vs baseline: 1.2401x; 1.2401x over previous
import functools

import jax
import jax.numpy as jnp
from jax import lax
from jax.experimental import pallas as pl
from jax.experimental.pallas import tpu as pltpu
from jax.experimental.pallas import tpu_sc as plsc

N = 10000
E = 320000
D = 128
NCORE = 2
NSUB = 16
NW = NCORE * NSUB
CH = 128
NCHUNK = E // CH
NP = 10240
NPH = NP // 16
BS = 640
NBLK = E // BS
F32 = jnp.float32

_mesh = plsc.VectorSubcoreMesh(core_axis_name="c", subcore_axis_name="s")


def _worker(c, s):
    return s * NCORE + c


def _degs_body(dst_hbm, deg_hbm, idx_d, hist, semA):
    c = lax.axis_index("c")
    s = lax.axis_index("s")
    w = _worker(c, s)

    def zrow(i, _):
        hist[pl.ds(i * 16, 16)] = jnp.zeros((16,), F32)
        return 0

    lax.fori_loop(0, NP // 16, zrow, 0)
    nch = jnp.int32(NCHUNK // NW) + jnp.where(w < NCHUNK % NW, 1, 0).astype(jnp.int32)
    ones16 = jnp.ones((16,), F32)

    def chunk(k, _):
        base = (w + NW * k) * CH
        pltpu.async_copy(dst_hbm.at[pl.ds(base, CH)], idx_d, semA).wait()
        for g in range(8):
            idx16 = idx_d[pl.ds(g * 16, 16)]
            plsc.addupdate_scatter(hist, (idx16,), ones16)
        return 0

    lax.fori_loop(0, nch, chunk, 0)
    pltpu.async_copy(hist, deg_hbm.at[pl.ds(w * NP, NP)], semA).wait()


_degs = functools.partial(
    pl.kernel,
    out_type=jax.ShapeDtypeStruct((NW * NP,), F32),
    mesh=_mesh,
    compiler_params=pltpu.CompilerParams(needs_layout_passes=False),
    scratch_types=[
        pltpu.VMEM((CH,), jnp.int32),
        pltpu.VMEM((NP,), F32),
        pltpu.SemaphoreType.DMA,
    ],
)(_degs_body)


def _edge_body(ec_hbm, src_hbm, dst_hbm, tsrc_hbm, tdst_hbm,
               enew_hbm, msgt_hbm, est_hbm,
               idx_s, idx_d, ec_v, g_v, q_v, mt_v, st_v, sA, sB, sC):
    c = lax.axis_index("c")
    s = lax.axis_index("s")
    w = _worker(c, s)
    nch = jnp.int32(NCHUNK // NW) + jnp.where(w < NCHUNK % NW, 1, 0).astype(jnp.int32)
    zero16 = jnp.zeros((16,), F32)
    init = (zero16,) * 16
    iota16 = lax.broadcasted_iota(jnp.int32, (16,), 0)
    rows_g = [iota16 + g * 16 for g in range(8)]

    def chunk_a(k, accs):
        base = (w + NW * k) * CH
        pltpu.async_copy(src_hbm.at[pl.ds(base, CH)], idx_s, sA).wait()
        pltpu.async_copy(dst_hbm.at[pl.ds(base, CH)], idx_d, sB).wait()
        cp1 = pltpu.async_copy(ec_hbm.at[pl.ds(base, CH)], ec_v, sA)
        cp2 = pltpu.async_copy(tsrc_hbm.at[idx_s], g_v, sB)
        cp3 = pltpu.async_copy(tdst_hbm.at[idx_d], q_v, sC)
        cp1.wait()
        cp2.wait()
        cp3.wait()

        def row(r, a):
            col_r = jnp.full((16,), 0, jnp.int32) + r
            sums = []
            sqs = []
            for g in range(8):
                sl = pl.ds(g * 16, 16)
                en = ec_v[r, sl] + g_v[r, sl] + q_v[r, sl]
                ec_v[r, sl] = en
                sig = 1.0 / (1.0 + jnp.exp(-en))
                val = sig * g_v[r, pl.ds(D + g * 16, 16)]
                plsc.store_scatter(mt_v, (rows_g[g], col_r), val)
                sums.append(a[g] + en)
                sqs.append(a[8 + g] + en * en)
            return tuple(sums + sqs)

        accs = lax.fori_loop(0, CH, row, accs)
        pltpu.async_copy(ec_v, enew_hbm.at[pl.ds(base, CH)], sA).wait()
        pltpu.async_copy(mt_v, msgt_hbm.at[:, pl.ds(base, CH)], sB).wait()
        return accs

    accs = lax.fori_loop(0, nch, chunk_a, init)
    for g in range(8):
        st_v[0, pl.ds(g * 16, 16)] = accs[g]
        st_v[1, pl.ds(g * 16, 16)] = accs[8 + g]
    pltpu.async_copy(st_v, est_hbm.at[w], sC).wait()


_edge = functools.partial(
    pl.kernel,
    out_type=[
        jax.ShapeDtypeStruct((E, D), F32),
        jax.ShapeDtypeStruct((D, E), F32),
        jax.ShapeDtypeStruct((NW, 2, D), F32),
    ],
    mesh=_mesh,
    compiler_params=pltpu.CompilerParams(needs_layout_passes=False),
    scratch_types=[
        pltpu.VMEM((CH,), jnp.int32),
        pltpu.VMEM((CH,), jnp.int32),
        pltpu.VMEM((CH, D), F32),
        pltpu.VMEM((CH, 2 * D), F32),
        pltpu.VMEM((CH, D), F32),
        pltpu.VMEM((D, CH), F32),
        pltpu.VMEM((2, D), F32),
        pltpu.SemaphoreType.DMA,
        pltpu.SemaphoreType.DMA,
        pltpu.SemaphoreType.DMA,
    ],
)(_edge_body)


def _hagg_body(msgt_hbm, dst_hbm, hagg_hbm, idx_d, mt_v, hist, semA, semB):
    c = lax.axis_index("c")
    s = lax.axis_index("s")
    w = _worker(c, s)
    ch0 = (w // 2) * 8
    par = w % 2

    def zrow(i, _):
        for g in range(8):
            hist[i, pl.ds(g * 16, 16)] = jnp.zeros((16,), F32)
        return 0

    lax.fori_loop(0, NPH, zrow, 0)

    def block(k, _):
        bb = 2 * k + par
        base = bb * BS
        pltpu.async_copy(dst_hbm.at[pl.ds(base, BS)], idx_d, semA).wait()
        pltpu.async_copy(msgt_hbm.at[pl.ds(ch0, 8), pl.ds(base, BS)], mt_v, semB).wait()

        def grp(g, _2):
            sl = pl.ds(g * 16, 16)
            dst16 = idx_d[sl]
            row16 = lax.shift_right_logical(dst16, 4)
            colb = lax.shift_left(jnp.bitwise_and(dst16, 15), 3)
            for c8 in range(8):
                val = mt_v[c8, sl]
                plsc.addupdate_scatter(hist, (row16, colb + c8), val)
            return 0

        lax.fori_loop(0, BS // 16, grp, 0)
        return 0

    lax.fori_loop(0, NBLK // 2, block, 0)
    pltpu.async_copy(hist, hagg_hbm.at[w], semA).wait()


_hagg = functools.partial(
    pl.kernel,
    out_type=jax.ShapeDtypeStruct((NW, NPH, D), F32),
    mesh=_mesh,
    compiler_params=pltpu.CompilerParams(needs_layout_passes=False),
    scratch_types=[
        pltpu.VMEM((BS,), jnp.int32),
        pltpu.VMEM((8, BS), F32),
        pltpu.VMEM((NPH, D), F32),
        pltpu.SemaphoreType.DMA,
        pltpu.SemaphoreType.DMA,
    ],
)(_hagg_body)


def _k1_body(h_ref, wn_ref, wnb_ref, dg_ref, h0_ref, nm_ref):
    deg = jnp.maximum(jnp.sum(dg_ref[...], axis=0), 1.0)
    nm_ref[...] = lax.rsqrt(deg)
    h0_ref[...] = (
        jnp.dot(h_ref[...], wn_ref[...], preferred_element_type=F32) + wnb_ref[...]
    )


_k1 = pl.pallas_call(
    _k1_body,
    grid=(10,),
    in_specs=[
        pl.BlockSpec((1000, D), lambda i: (i, 0)),
        pl.BlockSpec((D, D), lambda i: (0, 0)),
        pl.BlockSpec((1, D), lambda i: (0, 0)),
        pl.BlockSpec((NW, 1000, 1), lambda i: (0, i, 0)),
    ],
    out_specs=[
        pl.BlockSpec((1000, D), lambda i: (i, 0)),
        pl.BlockSpec((1000, 1), lambda i: (i, 0)),
    ],
    out_shape=[
        jax.ShapeDtypeStruct((N, D), F32),
        jax.ShapeDtypeStruct((N, 1), F32),
    ],
)


def _k2_body(e_ref, we_ref, web_ref, cm_ref, cb_ref, e0_ref, ec_ref):
    e0 = jnp.dot(e_ref[...], we_ref[...], preferred_element_type=F32) + web_ref[...]
    e0_ref[...] = e0
    ec_ref[...] = jnp.dot(e0, cm_ref[...], preferred_element_type=F32) + cb_ref[...]


_k2 = pl.pallas_call(
    _k2_body,
    grid=(E // 512,),
    in_specs=[
        pl.BlockSpec((512, D), lambda i: (i, 0)),
        pl.BlockSpec((D, D), lambda i: (0, 0)),
        pl.BlockSpec((1, D), lambda i: (0, 0)),
        pl.BlockSpec((D, D), lambda i: (0, 0)),
        pl.BlockSpec((1, D), lambda i: (0, 0)),
    ],
    out_specs=[
        pl.BlockSpec((512, D), lambda i: (i, 0)),
        pl.BlockSpec((512, D), lambda i: (i, 0)),
    ],
    out_shape=[
        jax.ShapeDtypeStruct((E, D), F32),
        jax.ShapeDtypeStruct((E, D), F32),
    ],
)


def _knode_body(h_ref, nm_ref, a_ref, ab_ref, b_ref, bb_ref, d_ref, db_ref,
                em_ref, eb_ref, ah_ref, ts_ref, td_ref):
    hh = h_ref[...]
    ah_ref[...] = jnp.dot(hh, a_ref[...], preferred_element_type=F32) + ab_ref[...]
    dh = jnp.dot(hh, d_ref[...], preferred_element_type=F32) + db_ref[...]
    bh = (jnp.dot(hh, b_ref[...], preferred_element_type=F32) + bb_ref[...]) * nm_ref[...]
    ts_ref[:, 0:D] = dh
    ts_ref[:, D : 2 * D] = bh
    td_ref[...] = jnp.dot(hh, em_ref[...], preferred_element_type=F32) + eb_ref[...]


_knode = pl.pallas_call(
    _knode_body,
    grid=(10,),
    in_specs=[
        pl.BlockSpec((1000, D), lambda i: (i, 0)),
        pl.BlockSpec((1000, 1), lambda i: (i, 0)),
    ]
    + [
        spec
        for _ in range(4)
        for spec in (
            pl.BlockSpec((D, D), lambda i: (0, 0)),
            pl.BlockSpec((1, D), lambda i: (0, 0)),
        )
    ],
    out_specs=[
        pl.BlockSpec((1000, D), lambda i: (i, 0)),
        pl.BlockSpec((1000, 2 * D), lambda i: (i, 0)),
        pl.BlockSpec((1000, D), lambda i: (i, 0)),
    ],
    out_shape=[
        jax.ShapeDtypeStruct((N, D), F32),
        jax.ShapeDtypeStruct((N, 2 * D), F32),
        jax.ShapeDtypeStruct((N, D), F32),
    ],
)


def _kedge_body(en_ref, ep_ref, st_ref, ge_ref, be_ref, cm_ref, cb_ref,
                ei_ref, ec_ref):
    red = jnp.sum(st_ref[...], axis=0)
    mu = red[0] / E
    var = red[1] / E - mu * mu
    a = ge_ref[...] * lax.rsqrt(var + 1e-5)
    bb = be_ref[...] - mu * a
    ei = jax.nn.relu(en_ref[...] * a + bb) + ep_ref[...]
    ei_ref[...] = ei
    ec_ref[...] = jnp.dot(ei, cm_ref[...], preferred_element_type=F32) + cb_ref[...]


_kedge = pl.pallas_call(
    _kedge_body,
    grid=(E // 512,),
    in_specs=[
        pl.BlockSpec((512, D), lambda i: (i, 0)),
        pl.BlockSpec((512, D), lambda i: (i, 0)),
        pl.BlockSpec((NW, 2, D), lambda i: (0, 0, 0)),
        pl.BlockSpec((1, D), lambda i: (0, 0)),
        pl.BlockSpec((1, D), lambda i: (0, 0)),
        pl.BlockSpec((D, D), lambda i: (0, 0)),
        pl.BlockSpec((1, D), lambda i: (0, 0)),
    ],
    out_specs=[
        pl.BlockSpec((512, D), lambda i: (i, 0)),
        pl.BlockSpec((512, D), lambda i: (i, 0)),
    ],
    out_shape=[
        jax.ShapeDtypeStruct((E, D), F32),
        jax.ShapeDtypeStruct((E, D), F32),
    ],
)


def _knode2_body(ah_ref, hg_ref, nm_ref, h_ref, gh_ref, bhp_ref, o_ref):
    hn = ah_ref[...] + nm_ref[...] * (hg_ref[0] + hg_ref[1])
    mu = jnp.mean(hn, axis=0, keepdims=True)
    dlt = hn - mu
    var = jnp.mean(dlt * dlt, axis=0, keepdims=True)
    o_ref[...] = (
        jax.nn.relu(dlt * lax.rsqrt(var + 1e-5) * gh_ref[...] + bhp_ref[...])
        + h_ref[...]
    )


_knode2 = pl.pallas_call(
    _knode2_body,
    out_shape=jax.ShapeDtypeStruct((N, D), F32),
)


def _khead_body(h_ref, w1_ref, b1_ref, w2_ref, b2_ref, w3_ref, b3_ref, o_ref):
    hg = jnp.mean(h_ref[...], axis=0, keepdims=True)
    hg = jax.nn.relu(jnp.dot(hg, w1_ref[...], preferred_element_type=F32) + b1_ref[...])
    hg = jax.nn.relu(jnp.dot(hg, w2_ref[...], preferred_element_type=F32) + b2_ref[...])
    o_ref[...] = jnp.dot(hg, w3_ref[...], preferred_element_type=F32) + b3_ref[...]


def kernel(h, e, edge_index, Wn, Wn_b, We, We_b, A, A_b, Bm, B_b, Cm, C_b,
           Dm, D_b, Em, E_b, gh, bh, ge, be, W1, b1, W2, b2, W3, b3):
    src = edge_index[0]
    dst = edge_index[1]
    L = A.shape[0]

    deg_raw = _degs(dst)
    degs2 = deg_raw.reshape(NW, NP)[:, :N, None]
    h_cur, nm = _k1(h, Wn, Wn_b.reshape(1, D), degs2)
    e_cur, ec = _k2(e, We, We_b.reshape(1, D), Cm[0], C_b[0].reshape(1, D))
    enew = est = None
    for i in range(L):
        if i > 0:
            e_cur, ec = _kedge(
                enew, e_cur, est,
                ge[i - 1].reshape(1, D), be[i - 1].reshape(1, D),
                Cm[i], C_b[i].reshape(1, D),
            )
        ah, ts, td = _knode(
            h_cur, nm,
            A[i], A_b[i].reshape(1, D), Bm[i], B_b[i].reshape(1, D),
            Dm[i], D_b[i].reshape(1, D), Em[i], E_b[i].reshape(1, D),
        )
        enew, msgt, est = _edge(ec, src, dst, ts, td)
        hg = _hagg(msgt, dst)
        hgt = (hg.reshape(16, 2, NPH, 16, 8)
               .transpose(1, 2, 3, 0, 4)
               .reshape(2, NP, D)[:, :N])
        h_cur = _knode2(
            ah, hgt, nm, h_cur,
            gh[i].reshape(1, D), bh[i].reshape(1, D),
        )
    return pl.pallas_call(
        _khead_body,
        out_shape=jax.ShapeDtypeStruct((1, W3.shape[1]), F32),
    )(h_cur, W1, b1.reshape(1, -1), W2, b2.reshape(1, -1), W3, b3.reshape(1, -1))


# speedup vs baseline: 1.2496x; 1.0076x over previous
import functools

import jax
import jax.numpy as jnp
from jax import lax
from jax.experimental import pallas as pl
from jax.experimental.pallas import tpu as pltpu
from jax.experimental.pallas import tpu_sc as plsc

N = 10000
E = 320000
D = 128
NCORE = 2
NSUB = 16
NW = NCORE * NSUB
CH = 128
NCHUNK = E // CH
NP = 10240
NPH = NP // 16
BS = 640
NBLK = E // BS
F32 = jnp.float32

_mesh = plsc.VectorSubcoreMesh(core_axis_name="c", subcore_axis_name="s")


def _worker(c, s):
    return s * NCORE + c


def _degs_body(dst_hbm, deg_hbm, idx_d, hist, semA):
    c = lax.axis_index("c")
    s = lax.axis_index("s")
    w = _worker(c, s)

    def zrow(i, _):
        hist[pl.ds(i * 16, 16)] = jnp.zeros((16,), F32)
        return 0

    lax.fori_loop(0, NP // 16, zrow, 0)
    nch = jnp.int32(NCHUNK // NW) + jnp.where(w < NCHUNK % NW, 1, 0).astype(jnp.int32)
    ones16 = jnp.ones((16,), F32)

    def chunk(k, _):
        base = (w + NW * k) * CH
        pltpu.async_copy(dst_hbm.at[pl.ds(base, CH)], idx_d, semA).wait()
        for g in range(8):
            idx16 = idx_d[pl.ds(g * 16, 16)]
            plsc.addupdate_scatter(hist, (idx16,), ones16)
        return 0

    lax.fori_loop(0, nch, chunk, 0)
    pltpu.async_copy(hist, deg_hbm.at[pl.ds(w * NP, NP)], semA).wait()


_degs = functools.partial(
    pl.kernel,
    out_type=jax.ShapeDtypeStruct((NW * NP,), F32),
    mesh=_mesh,
    compiler_params=pltpu.CompilerParams(needs_layout_passes=False),
    scratch_types=[
        pltpu.VMEM((CH,), jnp.int32),
        pltpu.VMEM((NP,), F32),
        pltpu.SemaphoreType.DMA,
    ],
)(_degs_body)


def _make_edge(write_e):
    def body(ec_hbm, src_hbm, dst_hbm, tsrc_hbm, tdst_hbm, *rest):
        if write_e:
            enew_hbm, msgt_hbm, est_hbm = rest[:3]
            idx_s, idx_d, ec_v, g_v, q_v, mt_v, st_v, sA, sB, sC = rest[3:]
        else:
            msgt_hbm = rest[0]
            idx_s, idx_d, ec_v, g_v, q_v, mt_v, st_v, sA, sB, sC = rest[1:]
        c = lax.axis_index("c")
        s = lax.axis_index("s")
        w = _worker(c, s)
        exc = jnp.maximum(w - 17, 0)
        start = 80 * w - 4 * exc
        nb = jnp.where(w < 17, 20, 19)
        zero16 = jnp.zeros((16,), F32)
        init = (zero16,) * 16
        iota16 = lax.broadcasted_iota(jnp.int32, (16,), 0)
        rows_g = [iota16 + g * 16 for g in range(8)]

        def block(b, accs):
            bbase = (start + b * 4) * CH
            cpi = pltpu.async_copy(src_hbm.at[pl.ds(bbase, 4 * CH)], idx_s, sA)
            cpj = pltpu.async_copy(dst_hbm.at[pl.ds(bbase, 4 * CH)], idx_d, sB)
            cpi.wait()
            cpj.wait()
            for j in range(4):
                base = bbase + j * CH
                cp1 = pltpu.async_copy(ec_hbm.at[pl.ds(base, CH)], ec_v, sA)
                cp2 = pltpu.async_copy(
                    tsrc_hbm.at[idx_s.at[pl.ds(j * CH, CH)]], g_v, sB)
                cp3 = pltpu.async_copy(
                    tdst_hbm.at[idx_d.at[pl.ds(j * CH, CH)]], q_v, sC)
                cp1.wait()
                cp2.wait()
                cp3.wait()

                def row(r, a):
                    col_r = jnp.full((16,), 0, jnp.int32) + r
                    sums = []
                    sqs = []
                    for g in range(8):
                        sl = pl.ds(g * 16, 16)
                        en = ec_v[r, sl] + g_v[r, sl] + q_v[r, sl]
                        sig = 1.0 / (1.0 + jnp.exp(-en))
                        val = sig * g_v[r, pl.ds(D + g * 16, 16)]
                        plsc.store_scatter(mt_v, (rows_g[g], col_r), val)
                        if write_e:
                            ec_v[r, sl] = en
                            sums.append(a[g] + en)
                            sqs.append(a[8 + g] + en * en)
                    return tuple(sums + sqs) if write_e else a

                accs = lax.fori_loop(0, CH, row, accs)
                if write_e:
                    pltpu.async_copy(ec_v, enew_hbm.at[pl.ds(base, CH)], sA).wait()
                pltpu.async_copy(mt_v, msgt_hbm.at[:, pl.ds(base, CH)], sB).wait()
            return accs

        accs = lax.fori_loop(0, nb, block, init)
        if write_e:
            for g in range(8):
                st_v[0, pl.ds(g * 16, 16)] = accs[g]
                st_v[1, pl.ds(g * 16, 16)] = accs[8 + g]
            pltpu.async_copy(st_v, est_hbm.at[w], sC).wait()

    outs = []
    if write_e:
        outs.append(jax.ShapeDtypeStruct((E, D), F32))
    outs.append(jax.ShapeDtypeStruct((D, E), F32))
    if write_e:
        outs.append(jax.ShapeDtypeStruct((NW, 2, D), F32))
    return functools.partial(
        pl.kernel,
        out_type=outs,
        mesh=_mesh,
        compiler_params=pltpu.CompilerParams(needs_layout_passes=False),
        scratch_types=[
            pltpu.VMEM((4 * CH,), jnp.int32),
            pltpu.VMEM((4 * CH,), jnp.int32),
            pltpu.VMEM((CH, D), F32),
            pltpu.VMEM((CH, 2 * D), F32),
            pltpu.VMEM((CH, D), F32),
            pltpu.VMEM((D, CH), F32),
            pltpu.VMEM((2, D), F32),
            pltpu.SemaphoreType.DMA,
            pltpu.SemaphoreType.DMA,
            pltpu.SemaphoreType.DMA,
        ],
    )(body)


_edge = _make_edge(True)
_edge_last = _make_edge(False)


def _hagg_body(msgt_hbm, dst_hbm, hagg_hbm, idx_d, mt_v, hist, semA, semB):
    c = lax.axis_index("c")
    s = lax.axis_index("s")
    w = _worker(c, s)
    ch0 = (w // 2) * 8
    par = w % 2

    def zrow(i, _):
        for g in range(8):
            hist[i, pl.ds(g * 16, 16)] = jnp.zeros((16,), F32)
        return 0

    lax.fori_loop(0, NPH, zrow, 0)

    def block(k, _):
        bb = 2 * k + par
        base = bb * BS
        pltpu.async_copy(dst_hbm.at[pl.ds(base, BS)], idx_d, semA).wait()
        pltpu.async_copy(msgt_hbm.at[pl.ds(ch0, 8), pl.ds(base, BS)], mt_v, semB).wait()

        def grp(g, _2):
            sl = pl.ds(g * 16, 16)
            dst16 = idx_d[sl]
            row16 = lax.shift_right_logical(dst16, 4)
            colb = lax.shift_left(jnp.bitwise_and(dst16, 15), 3)
            for c8 in range(8):
                val = mt_v[c8, sl]
                plsc.addupdate_scatter(hist, (row16, colb + c8), val)
            return 0

        lax.fori_loop(0, BS // 16, grp, 0)
        return 0

    lax.fori_loop(0, NBLK // 2, block, 0)
    pltpu.async_copy(hist, hagg_hbm.at[w], semA).wait()


_hagg = functools.partial(
    pl.kernel,
    out_type=jax.ShapeDtypeStruct((NW, NPH, D), F32),
    mesh=_mesh,
    compiler_params=pltpu.CompilerParams(needs_layout_passes=False),
    scratch_types=[
        pltpu.VMEM((BS,), jnp.int32),
        pltpu.VMEM((8, BS), F32),
        pltpu.VMEM((NPH, D), F32),
        pltpu.SemaphoreType.DMA,
        pltpu.SemaphoreType.DMA,
    ],
)(_hagg_body)


def _k1_body(h_ref, wn_ref, wnb_ref, dg_ref, h0_ref, nm_ref):
    deg = jnp.maximum(jnp.sum(dg_ref[...], axis=0), 1.0)
    nm_ref[...] = lax.rsqrt(deg)
    h0_ref[...] = (
        jnp.dot(h_ref[...], wn_ref[...], preferred_element_type=F32) + wnb_ref[...]
    )


_k1 = pl.pallas_call(
    _k1_body,
    grid=(10,),
    in_specs=[
        pl.BlockSpec((1000, D), lambda i: (i, 0)),
        pl.BlockSpec((D, D), lambda i: (0, 0)),
        pl.BlockSpec((1, D), lambda i: (0, 0)),
        pl.BlockSpec((NW, 1000, 1), lambda i: (0, i, 0)),
    ],
    out_specs=[
        pl.BlockSpec((1000, D), lambda i: (i, 0)),
        pl.BlockSpec((1000, 1), lambda i: (i, 0)),
    ],
    out_shape=[
        jax.ShapeDtypeStruct((N, D), F32),
        jax.ShapeDtypeStruct((N, 1), F32),
    ],
)


def _k2_body(e_ref, we_ref, web_ref, cm_ref, cb_ref, e0_ref, ec_ref):
    e0 = jnp.dot(e_ref[...], we_ref[...], preferred_element_type=F32) + web_ref[...]
    e0_ref[...] = e0
    ec_ref[...] = jnp.dot(e0, cm_ref[...], preferred_element_type=F32) + cb_ref[...]


_k2 = pl.pallas_call(
    _k2_body,
    grid=(E // 512,),
    in_specs=[
        pl.BlockSpec((512, D), lambda i: (i, 0)),
        pl.BlockSpec((D, D), lambda i: (0, 0)),
        pl.BlockSpec((1, D), lambda i: (0, 0)),
        pl.BlockSpec((D, D), lambda i: (0, 0)),
        pl.BlockSpec((1, D), lambda i: (0, 0)),
    ],
    out_specs=[
        pl.BlockSpec((512, D), lambda i: (i, 0)),
        pl.BlockSpec((512, D), lambda i: (i, 0)),
    ],
    out_shape=[
        jax.ShapeDtypeStruct((E, D), F32),
        jax.ShapeDtypeStruct((E, D), F32),
    ],
)


def _knode_body(h_ref, nm_ref, a_ref, ab_ref, b_ref, bb_ref, d_ref, db_ref,
                em_ref, eb_ref, ah_ref, ts_ref, td_ref):
    hh = h_ref[...]
    ah_ref[...] = jnp.dot(hh, a_ref[...], preferred_element_type=F32) + ab_ref[...]
    dh = jnp.dot(hh, d_ref[...], preferred_element_type=F32) + db_ref[...]
    bh = (jnp.dot(hh, b_ref[...], preferred_element_type=F32) + bb_ref[...]) * nm_ref[...]
    ts_ref[:, 0:D] = dh
    ts_ref[:, D : 2 * D] = bh
    td_ref[...] = jnp.dot(hh, em_ref[...], preferred_element_type=F32) + eb_ref[...]


_knode = pl.pallas_call(
    _knode_body,
    grid=(10,),
    in_specs=[
        pl.BlockSpec((1000, D), lambda i: (i, 0)),
        pl.BlockSpec((1000, 1), lambda i: (i, 0)),
    ]
    + [
        spec
        for _ in range(4)
        for spec in (
            pl.BlockSpec((D, D), lambda i: (0, 0)),
            pl.BlockSpec((1, D), lambda i: (0, 0)),
        )
    ],
    out_specs=[
        pl.BlockSpec((1000, D), lambda i: (i, 0)),
        pl.BlockSpec((1000, 2 * D), lambda i: (i, 0)),
        pl.BlockSpec((1000, D), lambda i: (i, 0)),
    ],
    out_shape=[
        jax.ShapeDtypeStruct((N, D), F32),
        jax.ShapeDtypeStruct((N, 2 * D), F32),
        jax.ShapeDtypeStruct((N, D), F32),
    ],
)


def _kedge_body(en_ref, ep_ref, st_ref, ge_ref, be_ref, cm_ref, cb_ref,
                ei_ref, ec_ref):
    red = jnp.sum(st_ref[...], axis=0)
    mu = red[0] / E
    var = red[1] / E - mu * mu
    a = ge_ref[...] * lax.rsqrt(var + 1e-5)
    bb = be_ref[...] - mu * a
    ei = jax.nn.relu(en_ref[...] * a + bb) + ep_ref[...]
    ei_ref[...] = ei
    ec_ref[...] = jnp.dot(ei, cm_ref[...], preferred_element_type=F32) + cb_ref[...]


_kedge = pl.pallas_call(
    _kedge_body,
    grid=(E // 512,),
    in_specs=[
        pl.BlockSpec((512, D), lambda i: (i, 0)),
        pl.BlockSpec((512, D), lambda i: (i, 0)),
        pl.BlockSpec((NW, 2, D), lambda i: (0, 0, 0)),
        pl.BlockSpec((1, D), lambda i: (0, 0)),
        pl.BlockSpec((1, D), lambda i: (0, 0)),
        pl.BlockSpec((D, D), lambda i: (0, 0)),
        pl.BlockSpec((1, D), lambda i: (0, 0)),
    ],
    out_specs=[
        pl.BlockSpec((512, D), lambda i: (i, 0)),
        pl.BlockSpec((512, D), lambda i: (i, 0)),
    ],
    out_shape=[
        jax.ShapeDtypeStruct((E, D), F32),
        jax.ShapeDtypeStruct((E, D), F32),
    ],
)


def _knode2_body(ah_ref, hg_ref, nm_ref, h_ref, gh_ref, bhp_ref, o_ref):
    hn = ah_ref[...] + nm_ref[...] * (hg_ref[0] + hg_ref[1])
    mu = jnp.mean(hn, axis=0, keepdims=True)
    dlt = hn - mu
    var = jnp.mean(dlt * dlt, axis=0, keepdims=True)
    o_ref[...] = (
        jax.nn.relu(dlt * lax.rsqrt(var + 1e-5) * gh_ref[...] + bhp_ref[...])
        + h_ref[...]
    )


_knode2 = pl.pallas_call(
    _knode2_body,
    out_shape=jax.ShapeDtypeStruct((N, D), F32),
)


def _khead_body(h_ref, w1_ref, b1_ref, w2_ref, b2_ref, w3_ref, b3_ref, o_ref):
    hg = jnp.mean(h_ref[...], axis=0, keepdims=True)
    hg = jax.nn.relu(jnp.dot(hg, w1_ref[...], preferred_element_type=F32) + b1_ref[...])
    hg = jax.nn.relu(jnp.dot(hg, w2_ref[...], preferred_element_type=F32) + b2_ref[...])
    o_ref[...] = jnp.dot(hg, w3_ref[...], preferred_element_type=F32) + b3_ref[...]


def kernel(h, e, edge_index, Wn, Wn_b, We, We_b, A, A_b, Bm, B_b, Cm, C_b,
           Dm, D_b, Em, E_b, gh, bh, ge, be, W1, b1, W2, b2, W3, b3):
    src = edge_index[0]
    dst = edge_index[1]
    L = A.shape[0]

    deg_raw = _degs(dst)
    degs2 = deg_raw.reshape(NW, NP)[:, :N, None]
    h_cur, nm = _k1(h, Wn, Wn_b.reshape(1, D), degs2)
    e_cur, ec = _k2(e, We, We_b.reshape(1, D), Cm[0], C_b[0].reshape(1, D))
    enew = est = None
    for i in range(L):
        if i > 0:
            e_cur, ec = _kedge(
                enew, e_cur, est,
                ge[i - 1].reshape(1, D), be[i - 1].reshape(1, D),
                Cm[i], C_b[i].reshape(1, D),
            )
        ah, ts, td = _knode(
            h_cur, nm,
            A[i], A_b[i].reshape(1, D), Bm[i], B_b[i].reshape(1, D),
            Dm[i], D_b[i].reshape(1, D), Em[i], E_b[i].reshape(1, D),
        )
        if i < L - 1:
            enew, msgt, est = _edge(ec, src, dst, ts, td)
        else:
            (msgt,) = _edge_last(ec, src, dst, ts, td)
        hg = _hagg(msgt, dst)
        hgt = (hg.reshape(16, 2, NPH, 16, 8)
               .transpose(1, 2, 3, 0, 4)
               .reshape(2, NP, D)[:, :N])
        h_cur = _knode2(
            ah, hgt, nm, h_cur,
            gh[i].reshape(1, D), bh[i].reshape(1, D),
        )
    return pl.pallas_call(
        _khead_body,
        out_shape=jax.ShapeDtypeStruct((1, W3.shape[1]), F32),
    )(h_cur, W1, b1.reshape(1, -1), W2, b2.reshape(1, -1), W3, b3.reshape(1, -1))
